# baseline (speedup 1.0000x reference)
import functools

import jax
import jax.numpy as jnp
from jax import lax
from jax.experimental import pallas as pl
from jax.experimental.pallas import tpu as pltpu
from jax.experimental.pallas import tpu_sc as plsc

N = 50000
E = 800000
F = 128
G = 256
EPS = 1e-5

NC, NS = 2, 16
EB = 128
EA = E + N
NBLK = 6656
EP = NBLK * EB
BPT = NBLK // NS
BPW = NBLK // (NC * NS)
NACC = 51200
RPT = NACC // NS
FB = RPT // EB
CW = 32
NCH = F // CW
UL = 4 * NACC - 3
DCW = 8
DUL = 16 * NACC - 15

_mesh = plsc.VectorSubcoreMesh(
    core_axis_name="c", subcore_axis_name="s", num_cores=NC, num_subcores=NS
)


def _flush(acc, outwin, flqv1, fbuf, sid, w):
    base = sid * RPT

    def body(fb, _):
        def fill(k, _):
            r = base + fb * EB + k * 16 + lax.broadcasted_iota(jnp.int32, (16,), 0)
            flqv1[0, pl.ds(k * 16, 16)] = (r >> 3) * (1024 // w) + (r & 7) * (128 // w)
            return 0
        lax.fori_loop(0, EB // 16, fill, 0)
        pltpu.sync_copy(acc.at[pl.ds(base + fb * EB, EB)], fbuf)
        pltpu.sync_copy(fbuf, outwin.at[flqv1.at[0]])
        return 0

    lax.fori_loop(0, FB, body, 0)


def _zero_acc(acc, zv, sid):
    def body(i, _):
        pltpu.sync_copy(zv, acc.at[pl.ds(sid * RPT + i * EB, EB)])
        return 0
    lax.fori_loop(0, FB, body, 0)


@functools.partial(
    pl.kernel,
    out_type=jax.ShapeDtypeStruct((NC, 16 * NACC, DCW), jnp.float32),
    mesh=_mesh,
    scratch_types=[
        pltpu.VMEM((BPW, EB), jnp.int32),
        pltpu.VMEM((1, EB), jnp.int32),
        pltpu.VMEM((EB, DCW), jnp.float32),
        pltpu.VMEM((EB, DCW), jnp.float32),
        pltpu.VMEM_SHARED((NACC, DCW), jnp.float32),
        pltpu.SemaphoreType.DMA,
    ],
    compiler_params=pltpu.CompilerParams(use_tc_tiling_on_sc=False),
)
def _deg_kernel(dst_hbm, zeros_hbm, ones_hbm, out_hbm,
                dstv, flqv1, ones_v, fbuf, accd, dsem):
    cid = lax.axis_index("c")
    sid = lax.axis_index("s")
    wid = cid * NS + sid
    pltpu.sync_copy(ones_hbm, ones_v)
    pltpu.sync_copy(zeros_hbm, fbuf)
    _zero_acc(accd, fbuf, sid)
    pltpu.sync_copy(dst_hbm.at[pl.ds(wid * BPW, BPW)], dstv)
    plsc.subcore_barrier()

    def body(t, _):
        for k in range(4):
            pltpu.async_copy(ones_v, accd.at[dstv.at[4 * t + k]], dsem, add=True)
        for k in range(4):
            pltpu.make_async_copy(ones_v, accd.at[dstv.at[0]], dsem).wait()
        return 0

    lax.fori_loop(0, BPW // 4, body, 0)
    plsc.subcore_barrier()
    _flush(accd, out_hbm.at[cid], flqv1, fbuf, sid, DCW)


@functools.partial(
    pl.kernel,
    out_type=jax.ShapeDtypeStruct((4 * NACC, CW), jnp.float32),
    mesh=_mesh,
    scratch_types=[
        [pltpu.VMEM((4, 2, EB), jnp.int32)] * 2,
        [pltpu.SemaphoreType.DMA] * 2,
        pltpu.VMEM((1, EB), jnp.int32),
        pltpu.VMEM((4 * EB, CW), jnp.float32),
        pltpu.SemaphoreType.DMA,
        pltpu.SemaphoreType.DMA,
        pltpu.VMEM((EB, CW), jnp.float32),
        pltpu.VMEM_SHARED((NACC, CW), jnp.float32),
    ],
    compiler_params=pltpu.CompilerParams(use_tc_tiling_on_sc=False),
)
def _prop_kernel(hs_hbm, edges_hbm, zeros_hbm, out_hbm,
                 ebufs, esems, flqv1, bigbuf, gsem, ssem, fbuf, acc):
    cid = lax.axis_index("c")
    sid = lax.axis_index("s")
    NG = BPT // 4
    gb = sid * NG

    def eload(t, e):
        pltpu.async_copy(edges_hbm.at[gb + t], ebufs[e], esems[e])

    def ewait(e):
        pltpu.make_async_copy(edges_hbm.at[0], ebufs[e], esems[e]).wait()

    for cp in range(NCH // NC):
        c = cid * (NCH // NC) + cp
        win = hs_hbm.at[pl.ds(c, UL)]
        pltpu.sync_copy(zeros_hbm, fbuf)
        _zero_acc(acc, fbuf, sid)
        plsc.subcore_barrier()

        def gfire(e, k):
            pltpu.async_copy(win.at[ebufs[e].at[k, 0]],
                             bigbuf.at[pl.ds(k * EB, EB)], gsem)

        def gwait_all():
            pltpu.make_async_copy(win.at[pl.ds(0, 4 * EB)], bigbuf, gsem).wait()

        def sfire(e, k):
            pltpu.async_copy(bigbuf.at[pl.ds(k * EB, EB)],
                             acc.at[ebufs[e].at[k, 1]], ssem, add=True)

        def swait_all():
            pltpu.make_async_copy(bigbuf, acc.at[pl.ds(0, 4 * EB)], ssem).wait()

        eload(0, 0)
        ewait(0)
        for k in range(4):
            gfire(0, k)

        def body2(u, _):
            for par in range(2):
                t = 2 * u + par
                pb, nb = par, 1 - par

                @pl.when(t < NG - 1)
                def _():
                    eload(t + 1, nb)

                gwait_all()
                for k in range(4):
                    sfire(pb, k)
                swait_all()

                @pl.when(t < NG - 1)
                def _():
                    ewait(nb)
                    for k in range(4):
                        gfire(nb, k)
            return 0

        lax.fori_loop(0, NG // 2, body2, 0)
        plsc.subcore_barrier()
        _flush(acc, out_hbm.at[pl.ds(c, UL)], flqv1, fbuf, sid, CW)
        plsc.subcore_barrier()


_RB = 2048
_NRB = NACC // _RB


def _mm1_body(x_ref, w_ref, dinv_ref, o_ref):
    h = lax.dot_general(x_ref[...], w_ref[...], (((1,), (0,)), ((), ())),
                        preferred_element_type=jnp.float32)
    o_ref[...] = h * dinv_ref[...]


def _mm1(x, w, dinv):
    return pl.pallas_call(
        _mm1_body,
        out_shape=jax.ShapeDtypeStruct((NACC, F), jnp.float32),
        grid=(_NRB,),
        in_specs=[
            pl.BlockSpec((_RB, F), lambda i: (i, 0)),
            pl.BlockSpec((F, F), lambda i: (0, 0)),
            pl.BlockSpec((_RB, 1), lambda i: (i, 0)),
        ],
        out_specs=pl.BlockSpec((_RB, F), lambda i: (i, 0)),
    )(x, w, dinv)


def _mm2_body(a_ref, dinv_ref, al_ref, be_ref, w_ref, o_ref):
    t = a_ref[...] * dinv_ref[...] * al_ref[...] + be_ref[...]
    t = jnp.maximum(t, 0.0)
    h = lax.dot_general(t, w_ref[...], (((1,), (0,)), ((), ())),
                        preferred_element_type=jnp.float32)
    o_ref[...] = h * dinv_ref[...]


def _mm2(acc, dinv, al, be, w):
    return pl.pallas_call(
        _mm2_body,
        out_shape=jax.ShapeDtypeStruct((NACC, F), jnp.float32),
        grid=(_NRB,),
        in_specs=[
            pl.BlockSpec((_RB, F), lambda i: (i, 0)),
            pl.BlockSpec((_RB, 1), lambda i: (i, 0)),
            pl.BlockSpec((1, F), lambda i: (0, 0)),
            pl.BlockSpec((1, F), lambda i: (0, 0)),
            pl.BlockSpec((F, F), lambda i: (0, 0)),
        ],
        out_specs=pl.BlockSpec((_RB, F), lambda i: (i, 0)),
    )(acc, dinv, al, be, w)


def _stats_body(a_ref, dinv_ref, s1_ref, s2_ref):
    i = pl.program_id(0)

    @pl.when(i == 0)
    def _():
        s1_ref[...] = jnp.zeros_like(s1_ref)
        s2_ref[...] = jnp.zeros_like(s2_ref)

    rows = i * _RB + lax.broadcasted_iota(jnp.int32, (_RB, 1), 0)
    y = jnp.where(rows < N, a_ref[...] * dinv_ref[...], 0.0)
    s1_ref[...] += jnp.sum(y, axis=0, keepdims=True)
    s2_ref[...] += jnp.sum(y * y, axis=0, keepdims=True)


def _stats(acc, dinv):
    return pl.pallas_call(
        _stats_body,
        out_shape=[jax.ShapeDtypeStruct((1, F), jnp.float32),
                   jax.ShapeDtypeStruct((1, F), jnp.float32)],
        grid=(_NRB,),
        in_specs=[
            pl.BlockSpec((_RB, F), lambda i: (i, 0)),
            pl.BlockSpec((_RB, 1), lambda i: (i, 0)),
        ],
        out_specs=[pl.BlockSpec((1, F), lambda i: (0, 0)),
                   pl.BlockSpec((1, F), lambda i: (0, 0))],
        compiler_params=pltpu.CompilerParams(
            dimension_semantics=("arbitrary",)),
    )(acc, dinv)


def _pool_body(a_ref, dinv_ref, al_ref, be_ref, b_ref, o_ref, sums, cnt):
    i = pl.program_id(0)

    @pl.when(i == 0)
    def _():
        sums[...] = jnp.zeros_like(sums)
        cnt[...] = jnp.zeros_like(cnt)

    t = a_ref[...] * dinv_ref[...] * al_ref[...] + be_ref[...]
    t = jnp.maximum(t, 0.0)
    b = b_ref[0, 0, :]
    onehot = (b[:, None] == lax.broadcasted_iota(jnp.int32, (_RB, G), 1)
              ).astype(jnp.float32)
    sums[...] += lax.dot_general(onehot, t, (((0,), (0,)), ((), ())),
                                 preferred_element_type=jnp.float32)
    cnt[...] += lax.dot_general(onehot, jnp.ones((_RB, 1), jnp.float32),
                                (((0,), (0,)), ((), ())),
                                preferred_element_type=jnp.float32)

    @pl.when(i == _NRB - 1)
    def _():
        o_ref[...] = sums[...] / jnp.maximum(cnt[...], 1.0)


def _pool(acc, dinv, al, be, batch3d):
    return pl.pallas_call(
        _pool_body,
        out_shape=jax.ShapeDtypeStruct((G, F), jnp.float32),
        grid=(_NRB,),
        in_specs=[
            pl.BlockSpec((_RB, F), lambda i: (i, 0)),
            pl.BlockSpec((_RB, 1), lambda i: (i, 0)),
            pl.BlockSpec((1, F), lambda i: (0, 0)),
            pl.BlockSpec((1, F), lambda i: (0, 0)),
            pl.BlockSpec((1, 1, _RB), lambda i: (i, 0, 0)),
        ],
        out_specs=pl.BlockSpec((G, F), lambda i: (0, 0)),
        scratch_shapes=[pltpu.VMEM((G, F), jnp.float32),
                        pltpu.VMEM((G, 1), jnp.float32)],
        compiler_params=pltpu.CompilerParams(
            dimension_semantics=("arbitrary",)),
    )(acc, dinv, al, be, batch3d)


def _pad128(w, g, be):
    fi, fo = w.shape
    wp = jnp.zeros((F, F), jnp.float32).at[:fi, :fo].set(w)
    gp = jnp.zeros((F,), jnp.float32).at[:fo].set(g)
    bp = jnp.zeros((F,), jnp.float32).at[:fo].set(be)
    return wp, gp, bp


def _uq(r, w):
    return (r >> 3) * (1024 // w) + (r & 7) * (128 // w)


def kernel(x, edge_index, batch, W1, b1, W2, b2, W3, b3,
           g1, be1, g2, be2, g3, be3):
    del b1, b2, b3
    sl = jnp.arange(N, dtype=jnp.int32)
    pad = EP - EA
    src = jnp.concatenate([edge_index[0], sl, jnp.zeros((pad,), jnp.int32)])
    dst = jnp.concatenate([edge_index[1], sl, jnp.full((pad,), N, jnp.int32)])
    srcq = _uq(src, CW).reshape(NBLK // 4, 4, EB)
    dst2d = dst.reshape(NBLK, EB)
    edges = jnp.stack([srcq, dst.reshape(NBLK // 4, 4, EB)], axis=2)
    zeros_h = jnp.zeros((EB, CW), jnp.float32)
    zeros_d = jnp.zeros((EB, DCW), jnp.float32)
    ones_h = jnp.ones((EB, DCW), jnp.float32)
    batch3d = jnp.concatenate(
        [batch, jnp.full((NACC - N,), G, jnp.int32)]).reshape(_NRB, 1, _RB)
    xp = jnp.concatenate([x, jnp.zeros((NACC - N, F), jnp.float32)])

    degp = _deg_kernel(dst2d, zeros_d, ones_h)
    d2 = degp.reshape(NC, NACC, F)
    deg = d2[0, :, 0] + d2[1, :, 0]
    dinv = lax.rsqrt(jnp.maximum(deg, 1.0)).reshape(NACC, 1)

    params = [_pad128(W1, g1, be1), _pad128(W2, g2, be2), _pad128(W3, g3, be3)]

    al = be_ = None
    accv = None
    for li, (wp, gp, bp) in enumerate(params):
        if li == 0:
            hs = _mm1(xp, wp, dinv)
        else:
            hs = _mm2(accv, dinv, al, be_, wp)
        acc_u = _prop_kernel(hs.reshape(4 * NACC, CW), edges, zeros_h)
        accv = acc_u.reshape(NACC, F)
        s1, s2 = _stats(accv, dinv)
        mu = s1 / N
        var = jnp.maximum(s2 / N - mu * mu, 0.0)
        alpha = gp.reshape(1, F) * lax.rsqrt(var + EPS)
        al = alpha
        be_ = bp.reshape(1, F) - mu * alpha

    return _pool(accv, dinv, al, be_, batch3d)


# speedup vs baseline: 1.0299x; 1.0299x over previous
import functools

import jax
import jax.numpy as jnp
from jax import lax
from jax.experimental import pallas as pl
from jax.experimental.pallas import tpu as pltpu
from jax.experimental.pallas import tpu_sc as plsc

N = 50000
E = 800000
F = 128
G = 256
EPS = 1e-5

NC, NS = 2, 16
EB = 128
EA = E + N
NBLK = 6656
EP = NBLK * EB
BPT = NBLK // NS
BPW = NBLK // (NC * NS)
NACC = 51200
RPT = NACC // NS
FB = RPT // EB
CW = 32
NCH = F // CW
UL = 4 * NACC - 3
DCW = 8
DUL = 16 * NACC - 15

_mesh = plsc.VectorSubcoreMesh(
    core_axis_name="c", subcore_axis_name="s", num_cores=NC, num_subcores=NS
)


def _flush(acc, outwin, flqv1, fbuf, sid, w):
    base = sid * RPT

    def body(fb, _):
        def fill(k, _):
            r = base + fb * EB + k * 16 + lax.broadcasted_iota(jnp.int32, (16,), 0)
            flqv1[0, pl.ds(k * 16, 16)] = (r >> 3) * (1024 // w) + (r & 7) * (128 // w)
            return 0
        lax.fori_loop(0, EB // 16, fill, 0)
        pltpu.sync_copy(acc.at[pl.ds(base + fb * EB, EB)], fbuf)
        pltpu.sync_copy(fbuf, outwin.at[flqv1.at[0]])
        return 0

    lax.fori_loop(0, FB, body, 0)


def _zero_acc(acc, zeros_hbm, sid):
    pltpu.sync_copy(zeros_hbm, acc.at[pl.ds(sid * RPT, RPT)])


@functools.partial(
    pl.kernel,
    out_type=jax.ShapeDtypeStruct((NC, 16 * NACC, DCW), jnp.float32),
    mesh=_mesh,
    scratch_types=[
        pltpu.VMEM((BPW, EB), jnp.int32),
        pltpu.VMEM((1, EB), jnp.int32),
        pltpu.VMEM((EB, DCW), jnp.float32),
        pltpu.VMEM((EB, DCW), jnp.float32),
        pltpu.VMEM_SHARED((NACC, DCW), jnp.float32),
        pltpu.SemaphoreType.DMA,
    ],
    compiler_params=pltpu.CompilerParams(use_tc_tiling_on_sc=False),
)
def _deg_kernel(dst_hbm, zeros_hbm, ones_hbm, out_hbm,
                dstv, flqv1, ones_v, fbuf, accd, dsem):
    cid = lax.axis_index("c")
    sid = lax.axis_index("s")
    wid = cid * NS + sid
    pltpu.sync_copy(ones_hbm, ones_v)
    _zero_acc(accd, zeros_hbm, sid)
    pltpu.sync_copy(dst_hbm.at[pl.ds(wid * BPW, BPW)], dstv)
    plsc.subcore_barrier()

    def body(t, _):
        for k in range(4):
            pltpu.async_copy(ones_v, accd.at[dstv.at[4 * t + k]], dsem, add=True)
        for k in range(4):
            pltpu.make_async_copy(ones_v, accd.at[dstv.at[0]], dsem).wait()
        return 0

    lax.fori_loop(0, BPW // 4, body, 0)
    plsc.subcore_barrier()
    _flush(accd, out_hbm.at[cid], flqv1, fbuf, sid, DCW)


@functools.partial(
    pl.kernel,
    out_type=jax.ShapeDtypeStruct((4 * NACC, CW), jnp.float32),
    mesh=_mesh,
    scratch_types=[
        [pltpu.VMEM((4, 2, EB), jnp.int32)] * 2,
        [pltpu.SemaphoreType.DMA] * 2,
        pltpu.VMEM((1, EB), jnp.int32),
        [pltpu.VMEM((EB, CW), jnp.float32)] * 4,
        [pltpu.SemaphoreType.DMA] * 4,
        [pltpu.SemaphoreType.DMA] * 4,
        pltpu.VMEM((EB, CW), jnp.float32),
        pltpu.VMEM_SHARED((NACC, CW), jnp.float32),
    ],
    compiler_params=pltpu.CompilerParams(use_tc_tiling_on_sc=False),
)
def _prop_kernel(hs_hbm, edges_hbm, zeros_hbm, out_hbm,
                 ebufs, esems, flqv1, bufs, gsems, ssems, fbuf, acc):
    cid = lax.axis_index("c")
    sid = lax.axis_index("s")
    NG = BPT // 4
    gb = sid * NG

    def eload(t, e):
        pltpu.async_copy(edges_hbm.at[gb + t], ebufs[e], esems[e])

    def ewait(e):
        pltpu.make_async_copy(edges_hbm.at[0], ebufs[e], esems[e]).wait()

    for cp in range(NCH // NC):
        c = cid * (NCH // NC) + cp
        win = hs_hbm.at[pl.ds(c, UL)]
        _zero_acc(acc, zeros_hbm, sid)
        plsc.subcore_barrier()

        def gfire(e, k):
            pltpu.async_copy(win.at[ebufs[e].at[k, 0]], bufs[k], gsems[k])

        def gwait(k):
            pltpu.make_async_copy(win.at[pl.ds(0, EB)], bufs[k], gsems[k]).wait()

        def sfire(e, k):
            pltpu.async_copy(bufs[k], acc.at[ebufs[e].at[k, 1]], ssems[k],
                             add=True)

        def swait(k):
            pltpu.make_async_copy(bufs[k], acc.at[pl.ds(0, EB)], ssems[k]).wait()

        eload(0, 0)
        ewait(0)
        for k in range(4):
            gfire(0, k)

        def body2(u, _):
            for par in range(2):
                t = 2 * u + par
                pb, nb = par, 1 - par

                @pl.when(t < NG - 1)
                def _():
                    eload(t + 1, nb)

                for k in range(4):
                    gwait(k)
                    sfire(pb, k)
                for k in range(4):
                    swait(k)

                @pl.when(t < NG - 1)
                def _():
                    ewait(nb)
                    for k in range(4):
                        gfire(nb, k)
            return 0

        lax.fori_loop(0, NG // 2, body2, 0)
        plsc.subcore_barrier()
        _flush(acc, out_hbm.at[pl.ds(c, UL)], flqv1, fbuf, sid, CW)
        plsc.subcore_barrier()


_RB = 2048
_NRB = NACC // _RB


def _mm1_body(x_ref, w_ref, dinv_ref, o_ref):
    h = lax.dot_general(x_ref[...], w_ref[...], (((1,), (0,)), ((), ())),
                        preferred_element_type=jnp.float32)
    o_ref[...] = h * dinv_ref[...]


def _mm1(x, w, dinv):
    return pl.pallas_call(
        _mm1_body,
        out_shape=jax.ShapeDtypeStruct((NACC, F), jnp.float32),
        grid=(_NRB,),
        in_specs=[
            pl.BlockSpec((_RB, F), lambda i: (i, 0)),
            pl.BlockSpec((F, F), lambda i: (0, 0)),
            pl.BlockSpec((_RB, 1), lambda i: (i, 0)),
        ],
        out_specs=pl.BlockSpec((_RB, F), lambda i: (i, 0)),
    )(x, w, dinv)


def _mm2_body(a_ref, dinv_ref, al_ref, be_ref, w_ref, o_ref):
    t = a_ref[...] * dinv_ref[...] * al_ref[...] + be_ref[...]
    t = jnp.maximum(t, 0.0)
    h = lax.dot_general(t, w_ref[...], (((1,), (0,)), ((), ())),
                        preferred_element_type=jnp.float32)
    o_ref[...] = h * dinv_ref[...]


def _mm2(acc, dinv, al, be, w):
    return pl.pallas_call(
        _mm2_body,
        out_shape=jax.ShapeDtypeStruct((NACC, F), jnp.float32),
        grid=(_NRB,),
        in_specs=[
            pl.BlockSpec((_RB, F), lambda i: (i, 0)),
            pl.BlockSpec((_RB, 1), lambda i: (i, 0)),
            pl.BlockSpec((1, F), lambda i: (0, 0)),
            pl.BlockSpec((1, F), lambda i: (0, 0)),
            pl.BlockSpec((F, F), lambda i: (0, 0)),
        ],
        out_specs=pl.BlockSpec((_RB, F), lambda i: (i, 0)),
    )(acc, dinv, al, be, w)


def _stats_body(a_ref, dinv_ref, s1_ref, s2_ref):
    i = pl.program_id(0)

    @pl.when(i == 0)
    def _():
        s1_ref[...] = jnp.zeros_like(s1_ref)
        s2_ref[...] = jnp.zeros_like(s2_ref)

    rows = i * _RB + lax.broadcasted_iota(jnp.int32, (_RB, 1), 0)
    y = jnp.where(rows < N, a_ref[...] * dinv_ref[...], 0.0)
    s1_ref[...] += jnp.sum(y, axis=0, keepdims=True)
    s2_ref[...] += jnp.sum(y * y, axis=0, keepdims=True)


def _stats(acc, dinv):
    return pl.pallas_call(
        _stats_body,
        out_shape=[jax.ShapeDtypeStruct((1, F), jnp.float32),
                   jax.ShapeDtypeStruct((1, F), jnp.float32)],
        grid=(_NRB,),
        in_specs=[
            pl.BlockSpec((_RB, F), lambda i: (i, 0)),
            pl.BlockSpec((_RB, 1), lambda i: (i, 0)),
        ],
        out_specs=[pl.BlockSpec((1, F), lambda i: (0, 0)),
                   pl.BlockSpec((1, F), lambda i: (0, 0))],
        compiler_params=pltpu.CompilerParams(
            dimension_semantics=("arbitrary",)),
    )(acc, dinv)


def _pool_body(a_ref, dinv_ref, al_ref, be_ref, b_ref, o_ref, sums, cnt):
    i = pl.program_id(0)

    @pl.when(i == 0)
    def _():
        sums[...] = jnp.zeros_like(sums)
        cnt[...] = jnp.zeros_like(cnt)

    t = a_ref[...] * dinv_ref[...] * al_ref[...] + be_ref[...]
    t = jnp.maximum(t, 0.0)
    b = b_ref[0, 0, :]
    onehot = (b[:, None] == lax.broadcasted_iota(jnp.int32, (_RB, G), 1)
              ).astype(jnp.float32)
    sums[...] += lax.dot_general(onehot, t, (((0,), (0,)), ((), ())),
                                 preferred_element_type=jnp.float32)
    cnt[...] += lax.dot_general(onehot, jnp.ones((_RB, 1), jnp.float32),
                                (((0,), (0,)), ((), ())),
                                preferred_element_type=jnp.float32)

    @pl.when(i == _NRB - 1)
    def _():
        o_ref[...] = sums[...] / jnp.maximum(cnt[...], 1.0)


def _pool(acc, dinv, al, be, batch3d):
    return pl.pallas_call(
        _pool_body,
        out_shape=jax.ShapeDtypeStruct((G, F), jnp.float32),
        grid=(_NRB,),
        in_specs=[
            pl.BlockSpec((_RB, F), lambda i: (i, 0)),
            pl.BlockSpec((_RB, 1), lambda i: (i, 0)),
            pl.BlockSpec((1, F), lambda i: (0, 0)),
            pl.BlockSpec((1, F), lambda i: (0, 0)),
            pl.BlockSpec((1, 1, _RB), lambda i: (i, 0, 0)),
        ],
        out_specs=pl.BlockSpec((G, F), lambda i: (0, 0)),
        scratch_shapes=[pltpu.VMEM((G, F), jnp.float32),
                        pltpu.VMEM((G, 1), jnp.float32)],
        compiler_params=pltpu.CompilerParams(
            dimension_semantics=("arbitrary",)),
    )(acc, dinv, al, be, batch3d)


def _pad128(w, g, be):
    fi, fo = w.shape
    wp = jnp.zeros((F, F), jnp.float32).at[:fi, :fo].set(w)
    gp = jnp.zeros((F,), jnp.float32).at[:fo].set(g)
    bp = jnp.zeros((F,), jnp.float32).at[:fo].set(be)
    return wp, gp, bp


def _uq(r, w):
    return (r >> 3) * (1024 // w) + (r & 7) * (128 // w)


def kernel(x, edge_index, batch, W1, b1, W2, b2, W3, b3,
           g1, be1, g2, be2, g3, be3):
    del b1, b2, b3
    sl = jnp.arange(N, dtype=jnp.int32)
    pad = EP - EA
    src = jnp.concatenate([edge_index[0], sl, jnp.zeros((pad,), jnp.int32)])
    dst = jnp.concatenate([edge_index[1], sl, jnp.full((pad,), N, jnp.int32)])
    srcq = _uq(src, CW).reshape(NBLK // 4, 4, EB)
    dst2d = dst.reshape(NBLK, EB)
    edges = jnp.stack([srcq, dst.reshape(NBLK // 4, 4, EB)], axis=2)
    zeros_h = jnp.zeros((RPT, CW), jnp.float32)
    zeros_d = jnp.zeros((RPT, DCW), jnp.float32)
    ones_h = jnp.ones((EB, DCW), jnp.float32)
    batch3d = jnp.concatenate(
        [batch, jnp.full((NACC - N,), G, jnp.int32)]).reshape(_NRB, 1, _RB)
    xp = jnp.concatenate([x, jnp.zeros((NACC - N, F), jnp.float32)])

    degp = _deg_kernel(dst2d, zeros_d, ones_h)
    d2 = degp.reshape(NC, NACC, F)
    deg = d2[0, :, 0] + d2[1, :, 0]
    dinv = lax.rsqrt(jnp.maximum(deg, 1.0)).reshape(NACC, 1)

    params = [_pad128(W1, g1, be1), _pad128(W2, g2, be2), _pad128(W3, g3, be3)]

    al = be_ = None
    accv = None
    for li, (wp, gp, bp) in enumerate(params):
        if li == 0:
            hs = _mm1(xp, wp, dinv)
        else:
            hs = _mm2(accv, dinv, al, be_, wp)
        acc_u = _prop_kernel(hs.reshape(4 * NACC, CW), edges, zeros_h)
        accv = acc_u.reshape(NACC, F)
        s1, s2 = _stats(accv, dinv)
        mu = s1 / N
        var = jnp.maximum(s2 / N - mu * mu, 0.0)
        alpha = gp.reshape(1, F) * lax.rsqrt(var + EPS)
        al = alpha
        be_ = bp.reshape(1, F) - mu * alpha

    return _pool(accv, dinv, al, be_, batch3d)


# speedup vs baseline: 1.0358x; 1.0058x over previous
import functools

import jax
import jax.numpy as jnp
from jax import lax
from jax.experimental import pallas as pl
from jax.experimental.pallas import tpu as pltpu
from jax.experimental.pallas import tpu_sc as plsc

N = 50000
E = 800000
F = 128
G = 256
EPS = 1e-5

NC, NS = 2, 16
EB = 128
EA = E + N
NBLK = 6656
EP = NBLK * EB
BPT = NBLK // NS
BPW = NBLK // (NC * NS)
NACC = 51200
RPT = NACC // NS
FB = RPT // EB
CW = 32
NCH = F // CW
UL = 4 * NACC - 3
DCW = 8
DUL = 16 * NACC - 15

_mesh = plsc.VectorSubcoreMesh(
    core_axis_name="c", subcore_axis_name="s", num_cores=NC, num_subcores=NS
)


def _flush(acc, outwin, flqv1, fbuf, sid, w):
    base = sid * RPT

    def body(fb, _):
        def fill(k, _):
            r = base + fb * EB + k * 16 + lax.broadcasted_iota(jnp.int32, (16,), 0)
            flqv1[0, pl.ds(k * 16, 16)] = (r >> 3) * (1024 // w) + (r & 7) * (128 // w)
            return 0
        lax.fori_loop(0, EB // 16, fill, 0)
        pltpu.sync_copy(acc.at[pl.ds(base + fb * EB, EB)], fbuf)
        pltpu.sync_copy(fbuf, outwin.at[flqv1.at[0]])
        return 0

    lax.fori_loop(0, FB, body, 0)


def _zero_acc(acc, zv, sid):
    def body(i, _):
        pltpu.sync_copy(zv, acc.at[pl.ds(sid * RPT + i * EB, EB)])
        return 0
    lax.fori_loop(0, FB, body, 0)


@functools.partial(
    pl.kernel,
    out_type=jax.ShapeDtypeStruct((NC, 16 * NACC, DCW), jnp.float32),
    mesh=_mesh,
    scratch_types=[
        pltpu.VMEM((BPW, EB), jnp.int32),
        pltpu.VMEM((1, EB), jnp.int32),
        pltpu.VMEM((EB, DCW), jnp.float32),
        pltpu.VMEM((EB, DCW), jnp.float32),
        pltpu.VMEM_SHARED((NACC, DCW), jnp.float32),
        pltpu.SemaphoreType.DMA,
    ],
    compiler_params=pltpu.CompilerParams(use_tc_tiling_on_sc=False),
)
def _deg_kernel(dst_hbm, zeros_hbm, ones_hbm, out_hbm,
                dstv, flqv1, ones_v, fbuf, accd, dsem):
    cid = lax.axis_index("c")
    sid = lax.axis_index("s")
    wid = cid * NS + sid
    pltpu.sync_copy(ones_hbm, ones_v)
    pltpu.sync_copy(zeros_hbm, fbuf)
    _zero_acc(accd, fbuf, sid)
    pltpu.sync_copy(dst_hbm.at[pl.ds(wid * BPW, BPW)], dstv)
    plsc.subcore_barrier()

    def body(t, _):
        for k in range(4):
            pltpu.async_copy(ones_v, accd.at[dstv.at[4 * t + k]], dsem, add=True)
        for k in range(4):
            pltpu.make_async_copy(ones_v, accd.at[dstv.at[0]], dsem).wait()
        return 0

    lax.fori_loop(0, BPW // 4, body, 0)
    plsc.subcore_barrier()
    _flush(accd, out_hbm.at[cid], flqv1, fbuf, sid, DCW)


@functools.partial(
    pl.kernel,
    out_type=jax.ShapeDtypeStruct((4 * NACC, CW), jnp.float32),
    mesh=_mesh,
    scratch_types=[
        [pltpu.VMEM((4, 2, EB), jnp.int32)] * 2,
        [pltpu.SemaphoreType.DMA] * 2,
        pltpu.VMEM((1, EB), jnp.int32),
        [pltpu.VMEM((EB, CW), jnp.float32)] * 4,
        [pltpu.SemaphoreType.DMA] * 4,
        [pltpu.SemaphoreType.DMA] * 4,
        pltpu.VMEM((EB, CW), jnp.float32),
        pltpu.VMEM_SHARED((NACC, CW), jnp.float32),
    ],
    compiler_params=pltpu.CompilerParams(use_tc_tiling_on_sc=False),
)
def _prop_kernel(hs_hbm, edges_hbm, zeros_hbm, out_hbm,
                 ebufs, esems, flqv1, bufs, gsems, ssems, fbuf, acc):
    cid = lax.axis_index("c")
    sid = lax.axis_index("s")
    NG = BPT // 4
    gb = sid * NG

    def eload(t, e):
        pltpu.async_copy(edges_hbm.at[gb + t], ebufs[e], esems[e])

    def ewait(e):
        pltpu.make_async_copy(edges_hbm.at[0], ebufs[e], esems[e]).wait()

    for cp in range(NCH // NC):
        c = cid * (NCH // NC) + cp
        win = hs_hbm.at[pl.ds(c, UL)]
        pltpu.sync_copy(zeros_hbm, fbuf)
        _zero_acc(acc, fbuf, sid)
        plsc.subcore_barrier()

        def gfire(e, k):
            pltpu.async_copy(win.at[ebufs[e].at[k, 0]], bufs[k], gsems[k])

        def gwait(k):
            pltpu.make_async_copy(win.at[pl.ds(0, EB)], bufs[k], gsems[k]).wait()

        def sfire(e, k):
            pltpu.async_copy(bufs[k], acc.at[ebufs[e].at[k, 1]], ssems[k],
                             add=True)

        def swait(k):
            pltpu.make_async_copy(bufs[k], acc.at[pl.ds(0, EB)], ssems[k]).wait()

        eload(0, 0)
        ewait(0)
        for k in range(4):
            gfire(0, k)

        def body2(u, _):
            for par in range(2):
                t = 2 * u + par
                pb, nb = par, 1 - par

                @pl.when(t < NG - 1)
                def _():
                    eload(t + 1, nb)

                for k in range(4):
                    gwait(k)
                    sfire(pb, k)
                for k in range(4):
                    swait(k)

                @pl.when(t < NG - 1)
                def _():
                    ewait(nb)
                    for k in range(4):
                        gfire(nb, k)
            return 0

        lax.fori_loop(0, NG // 2, body2, 0)
        plsc.subcore_barrier()
        _flush(acc, out_hbm.at[pl.ds(c, UL)], flqv1, fbuf, sid, CW)
        plsc.subcore_barrier()


_RB = 2048
_NRB = NACC // _RB


def _mm1_body(x_ref, w_ref, dinv_ref, o_ref):
    h = lax.dot_general(x_ref[...], w_ref[...], (((1,), (0,)), ((), ())),
                        preferred_element_type=jnp.float32)
    o_ref[...] = h * dinv_ref[...]


def _mm1(x, w, dinv):
    return pl.pallas_call(
        _mm1_body,
        out_shape=jax.ShapeDtypeStruct((NACC, F), jnp.float32),
        grid=(_NRB,),
        in_specs=[
            pl.BlockSpec((_RB, F), lambda i: (i, 0)),
            pl.BlockSpec((F, F), lambda i: (0, 0)),
            pl.BlockSpec((_RB, 1), lambda i: (i, 0)),
        ],
        out_specs=pl.BlockSpec((_RB, F), lambda i: (i, 0)),
    )(x, w, dinv)


def _mm2_body(a_ref, dinv_ref, al_ref, be_ref, w_ref, o_ref):
    t = a_ref[...] * dinv_ref[...] * al_ref[...] + be_ref[...]
    t = jnp.maximum(t, 0.0)
    h = lax.dot_general(t, w_ref[...], (((1,), (0,)), ((), ())),
                        preferred_element_type=jnp.float32)
    o_ref[...] = h * dinv_ref[...]


def _mm2(acc, dinv, al, be, w):
    return pl.pallas_call(
        _mm2_body,
        out_shape=jax.ShapeDtypeStruct((NACC, F), jnp.float32),
        grid=(_NRB,),
        in_specs=[
            pl.BlockSpec((_RB, F), lambda i: (i, 0)),
            pl.BlockSpec((_RB, 1), lambda i: (i, 0)),
            pl.BlockSpec((1, F), lambda i: (0, 0)),
            pl.BlockSpec((1, F), lambda i: (0, 0)),
            pl.BlockSpec((F, F), lambda i: (0, 0)),
        ],
        out_specs=pl.BlockSpec((_RB, F), lambda i: (i, 0)),
    )(acc, dinv, al, be, w)


def _stats_body(a_ref, dinv_ref, s1_ref, s2_ref):
    i = pl.program_id(0)

    @pl.when(i == 0)
    def _():
        s1_ref[...] = jnp.zeros_like(s1_ref)
        s2_ref[...] = jnp.zeros_like(s2_ref)

    rows = i * _RB + lax.broadcasted_iota(jnp.int32, (_RB, 1), 0)
    y = jnp.where(rows < N, a_ref[...] * dinv_ref[...], 0.0)
    s1_ref[...] += jnp.sum(y, axis=0, keepdims=True)
    s2_ref[...] += jnp.sum(y * y, axis=0, keepdims=True)


def _stats(acc, dinv):
    return pl.pallas_call(
        _stats_body,
        out_shape=[jax.ShapeDtypeStruct((1, F), jnp.float32),
                   jax.ShapeDtypeStruct((1, F), jnp.float32)],
        grid=(_NRB,),
        in_specs=[
            pl.BlockSpec((_RB, F), lambda i: (i, 0)),
            pl.BlockSpec((_RB, 1), lambda i: (i, 0)),
        ],
        out_specs=[pl.BlockSpec((1, F), lambda i: (0, 0)),
                   pl.BlockSpec((1, F), lambda i: (0, 0))],
        compiler_params=pltpu.CompilerParams(
            dimension_semantics=("arbitrary",)),
    )(acc, dinv)


def _pool_body(a_ref, dinv_ref, al_ref, be_ref, b_ref, o_ref, sums, cnt):
    i = pl.program_id(0)

    @pl.when(i == 0)
    def _():
        sums[...] = jnp.zeros_like(sums)
        cnt[...] = jnp.zeros_like(cnt)

    t = a_ref[...] * dinv_ref[...] * al_ref[...] + be_ref[...]
    t = jnp.maximum(t, 0.0)
    b = b_ref[0, 0, :]
    onehot = (b[:, None] == lax.broadcasted_iota(jnp.int32, (_RB, G), 1)
              ).astype(jnp.float32)
    sums[...] += lax.dot_general(onehot, t, (((0,), (0,)), ((), ())),
                                 preferred_element_type=jnp.float32)
    cnt[...] += lax.dot_general(onehot, jnp.ones((_RB, 1), jnp.float32),
                                (((0,), (0,)), ((), ())),
                                preferred_element_type=jnp.float32)

    @pl.when(i == _NRB - 1)
    def _():
        o_ref[...] = sums[...] / jnp.maximum(cnt[...], 1.0)


def _pool(acc, dinv, al, be, batch3d):
    return pl.pallas_call(
        _pool_body,
        out_shape=jax.ShapeDtypeStruct((G, F), jnp.float32),
        grid=(_NRB,),
        in_specs=[
            pl.BlockSpec((_RB, F), lambda i: (i, 0)),
            pl.BlockSpec((_RB, 1), lambda i: (i, 0)),
            pl.BlockSpec((1, F), lambda i: (0, 0)),
            pl.BlockSpec((1, F), lambda i: (0, 0)),
            pl.BlockSpec((1, 1, _RB), lambda i: (i, 0, 0)),
        ],
        out_specs=pl.BlockSpec((G, F), lambda i: (0, 0)),
        scratch_shapes=[pltpu.VMEM((G, F), jnp.float32),
                        pltpu.VMEM((G, 1), jnp.float32)],
        compiler_params=pltpu.CompilerParams(
            dimension_semantics=("arbitrary",)),
    )(acc, dinv, al, be, batch3d)


def _pad128(w, g, be):
    fi, fo = w.shape
    wp = jnp.zeros((F, F), jnp.float32).at[:fi, :fo].set(w)
    gp = jnp.zeros((F,), jnp.float32).at[:fo].set(g)
    bp = jnp.zeros((F,), jnp.float32).at[:fo].set(be)
    return wp, gp, bp


def _uq(r, w):
    return (r >> 3) * (1024 // w) + (r & 7) * (128 // w)


def kernel(x, edge_index, batch, W1, b1, W2, b2, W3, b3,
           g1, be1, g2, be2, g3, be3):
    del b1, b2, b3
    sl = jnp.arange(N, dtype=jnp.int32)
    pad = EP - EA
    src = jnp.concatenate([edge_index[0], sl, jnp.zeros((pad,), jnp.int32)])
    dst = jnp.concatenate([edge_index[1], sl, jnp.full((pad,), N, jnp.int32)])
    srcq = _uq(src, CW).reshape(NBLK // 4, 4, EB)
    dst2d = dst.reshape(NBLK, EB)
    edges = jnp.stack([srcq, dst.reshape(NBLK // 4, 4, EB)], axis=2)
    zeros_h = jnp.zeros((EB, CW), jnp.float32)
    zeros_d = jnp.zeros((EB, DCW), jnp.float32)
    ones_h = jnp.ones((EB, DCW), jnp.float32)
    batch3d = jnp.concatenate(
        [batch, jnp.full((NACC - N,), G, jnp.int32)]).reshape(_NRB, 1, _RB)
    xp = jnp.concatenate([x, jnp.zeros((NACC - N, F), jnp.float32)])

    degp = _deg_kernel(dst2d, zeros_d, ones_h)
    d2 = degp.reshape(NC, NACC, F)
    deg = d2[0, :, 0] + d2[1, :, 0]
    dinv = lax.rsqrt(jnp.maximum(deg, 1.0)).reshape(NACC, 1)

    params = [_pad128(W1, g1, be1), _pad128(W2, g2, be2), _pad128(W3, g3, be3)]

    al = be_ = None
    accv = None
    for li, (wp, gp, bp) in enumerate(params):
        if li == 0:
            hs = _mm1(xp, wp, dinv)
        else:
            hs = _mm2(accv, dinv, al, be_, wp)
        acc_u = _prop_kernel(hs.reshape(4 * NACC, CW), edges, zeros_h)
        accv = acc_u.reshape(NACC, F)
        s1, s2 = _stats(accv, dinv)
        mu = s1 / N
        var = jnp.maximum(s2 / N - mu * mu, 0.0)
        alpha = gp.reshape(1, F) * lax.rsqrt(var + EPS)
        al = alpha
        be_ = bp.reshape(1, F) - mu * alpha

    return _pool(accv, dinv, al, be_, batch3d)
